# Initial kernel scaffold; baseline (speedup 1.0000x reference)
import jax
import jax.numpy as jnp
from jax import lax
from jax.experimental import pallas as pl
from jax.experimental.pallas import tpu as pltpu
from jax.experimental.pallas import tpu_sc as plsc

N = 10000
E = 320000
H = 64
NEG_SLOPE = 0.2
ETOT = E + N

NTILES = 32
NCHUNK = 81
CK = 128
EPT = NCHUNK * CK
EPAD = NTILES * EPT
RPT = N // 16


_BN = 2000


def _tc_a_body(x_ref, w_ref, asv_ref, adv_ref, h_ref, as_ref, ad_ref):
    h = jnp.dot(x_ref[...], w_ref[...], preferred_element_type=jnp.float32)
    h_ref[...] = h
    as_ref[...] = jnp.dot(h, asv_ref[...], preferred_element_type=jnp.float32)
    ad_ref[...] = jnp.dot(h, adv_ref[...], preferred_element_type=jnp.float32)


def _tc_a(x, W, a_src, a_dst):
    din = x.shape[1]
    return pl.pallas_call(
        _tc_a_body,
        grid=(N // _BN,),
        in_specs=[
            pl.BlockSpec((_BN, din), lambda i: (i, 0)),
            pl.BlockSpec((din, H), lambda i: (0, 0)),
            pl.BlockSpec((H, 1), lambda i: (0, 0)),
            pl.BlockSpec((H, 1), lambda i: (0, 0)),
        ],
        out_specs=[
            pl.BlockSpec((_BN, H), lambda i: (i, 0)),
            pl.BlockSpec((_BN, 1), lambda i: (i, 0)),
            pl.BlockSpec((_BN, 1), lambda i: (i, 0)),
        ],
        out_shape=[
            jax.ShapeDtypeStruct((N, H), jnp.float32),
            jax.ShapeDtypeStruct((N, 1), jnp.float32),
            jax.ShapeDtypeStruct((N, 1), jnp.float32),
        ],
    )(x, W, a_src.reshape(H, 1), a_dst.reshape(H, 1))


def _tc_b_body(acc_ref, den_ref, b_ref, w_ref, asv_ref, adv_ref,
               h_ref, as_ref, ad_ref):
    a = acc_ref[0] + acc_ref[1]
    d = den_ref[0, :, 0:1] + den_ref[1, :, 0:1]
    hag = a / (d + 1e-16) + b_ref[...]
    hag = jnp.maximum(hag, 0.0)
    h = jnp.dot(hag, w_ref[...], preferred_element_type=jnp.float32)
    h_ref[...] = h
    as_ref[...] = jnp.dot(h, asv_ref[...], preferred_element_type=jnp.float32)
    ad_ref[...] = jnp.dot(h, adv_ref[...], preferred_element_type=jnp.float32)


def _tc_b(acc, den, b, W, a_src, a_dst):
    return pl.pallas_call(
        _tc_b_body,
        grid=(N // _BN,),
        in_specs=[
            pl.BlockSpec((2, _BN, H), lambda i: (0, i, 0)),
            pl.BlockSpec((2, _BN, 16), lambda i: (0, i, 0)),
            pl.BlockSpec((1, H), lambda i: (0, 0)),
            pl.BlockSpec((H, H), lambda i: (0, 0)),
            pl.BlockSpec((H, 1), lambda i: (0, 0)),
            pl.BlockSpec((H, 1), lambda i: (0, 0)),
        ],
        out_specs=[
            pl.BlockSpec((_BN, H), lambda i: (i, 0)),
            pl.BlockSpec((_BN, 1), lambda i: (i, 0)),
            pl.BlockSpec((_BN, 1), lambda i: (i, 0)),
        ],
        out_shape=[
            jax.ShapeDtypeStruct((N, H), jnp.float32),
            jax.ShapeDtypeStruct((N, 1), jnp.float32),
            jax.ShapeDtypeStruct((N, 1), jnp.float32),
        ],
    )(acc, den, b.reshape(1, H), W, a_src.reshape(H, 1), a_dst.reshape(H, 1))


def _tc_c_body(acc_ref, den_ref, b_ref, w1_ref, b1_ref, w2_ref, b2_ref,
               out_ref):
    a = acc_ref[0] + acc_ref[1]
    d = den_ref[0, :, 0:1] + den_ref[1, :, 0:1]
    hag = a / (d + 1e-16) + b_ref[...]
    hag = jnp.maximum(hag, 0.0)
    h = jnp.dot(hag, w1_ref[...], preferred_element_type=jnp.float32)
    h = jnp.maximum(h + b1_ref[...], 0.0)
    out_ref[...] = jnp.dot(h, w2_ref[...],
                           preferred_element_type=jnp.float32) + b2_ref[...]


def _tc_c(acc, den, b, Wl1, bl1, Wl2, bl2):
    return pl.pallas_call(
        _tc_c_body,
        grid=(N // _BN,),
        in_specs=[
            pl.BlockSpec((2, _BN, H), lambda i: (0, i, 0)),
            pl.BlockSpec((2, _BN, 16), lambda i: (0, i, 0)),
            pl.BlockSpec((1, H), lambda i: (0, 0)),
            pl.BlockSpec((H, H), lambda i: (0, 0)),
            pl.BlockSpec((1, H), lambda i: (0, 0)),
            pl.BlockSpec((H, H), lambda i: (0, 0)),
            pl.BlockSpec((1, H), lambda i: (0, 0)),
        ],
        out_specs=pl.BlockSpec((_BN, H), lambda i: (i, 0)),
        out_shape=jax.ShapeDtypeStruct((N, H), jnp.float32),
    )(acc, den, b.reshape(1, H), Wl1, bl1.reshape(1, H), Wl2,
      bl2.reshape(1, H))


def _sc_body(src_hbm, dst_hbm, h_hbm, as_hbm, ad_hbm,
             acc_out, den_out,
             as_v, ad_v, sidx_v, didx_v, rows_v, wrow_v, zbuf_v,
             acc_sp, den_sp):
    cid = lax.axis_index("c")
    sid = lax.axis_index("s")
    wid = cid * 16 + sid

    pltpu.sync_copy(as_hbm, as_v)
    pltpu.sync_copy(ad_hbm, ad_v)
    pltpu.sync_copy(src_hbm.at[wid], sidx_v)
    pltpu.sync_copy(dst_hbm.at[wid], didx_v)

    zero16 = jnp.zeros((16,), jnp.float32)

    @pl.loop(0, 125)
    def _(i):
        for f in range(4):
            zbuf_v[i, pl.ds(16 * f, 16)] = zero16

    for r in range(5):
        pltpu.sync_copy(zbuf_v, acc_sp.at[pl.ds(sid * RPT + r * 125, 125)])
    for r in range(5):
        pltpu.sync_copy(zbuf_v.at[:, pl.ds(0, 16)],
                        den_sp.at[pl.ds(sid * RPT + r * 125, 125)])

    def _vmax(ref):
        def body(i, m):
            return jnp.maximum(m, ref[pl.ds(16 * i, 16)])
        m = lax.fori_loop(0, N // 16, body,
                          jnp.full((16,), -jnp.inf, jnp.float32))
        return lax.reduce_max_p.bind(m, axes=(0,))

    cmax = _vmax(as_v) + _vmax(ad_v)
    cshift = jnp.where(cmax >= 0.0, cmax, NEG_SLOPE * cmax)

    @pl.loop(0, CK)
    def _(i):
        wrow_v[i, :] = zero16

    plsc.subcore_barrier()

    lane = lax.iota(jnp.int32, (16,))
    zlane = jnp.zeros((16,), jnp.int32)
    base = wid * EPT

    @pl.loop(0, NCHUNK)
    def _chunk(j):
        pltpu.sync_copy(h_hbm.at[sidx_v.at[j]], rows_v)

        for g in range(8):
            si = sidx_v[j, pl.ds(16 * g, 16)]
            di = didx_v[j, pl.ds(16 * g, 16)]
            e = plsc.load_gather(as_v, [si]) + plsc.load_gather(ad_v, [di])
            e = jnp.where(e >= 0.0, e, NEG_SLOPE * e)
            w = jnp.exp(e - cshift)
            eid = base + j * CK + 16 * g + lane
            w = jnp.where(eid < ETOT, w, 0.0)
            plsc.store_scatter(wrow_v, [16 * g + lane, zlane], w)
            for i in range(16):
                s = w[i]
                row = 16 * g + i
                for f in range(4):
                    rows_v[row, pl.ds(16 * f, 16)] = (
                        rows_v[row, pl.ds(16 * f, 16)] * s)

        pltpu.sync_copy(rows_v, acc_sp.at[didx_v.at[j]], add=True)
        pltpu.sync_copy(wrow_v, den_sp.at[didx_v.at[j]], add=True)

    plsc.subcore_barrier()

    pltpu.sync_copy(acc_sp.at[pl.ds(sid * RPT, RPT)],
                    acc_out.at[cid, pl.ds(sid * RPT, RPT)])
    pltpu.sync_copy(den_sp.at[pl.ds(sid * RPT, RPT)],
                    den_out.at[cid, pl.ds(sid * RPT, RPT)])


def _sc_edge_pass(src3, dst3, h, as_, ad_):
    kern = pl.kernel(
        _sc_body,
        out_type=[
            jax.ShapeDtypeStruct((2, N, H), jnp.float32),
            jax.ShapeDtypeStruct((2, N, 16), jnp.float32),
        ],
        mesh=plsc.VectorSubcoreMesh(core_axis_name="c", subcore_axis_name="s"),
        scratch_types=[
            pltpu.VMEM((N,), jnp.float32),
            pltpu.VMEM((N,), jnp.float32),
            pltpu.VMEM((NCHUNK, CK), jnp.int32),
            pltpu.VMEM((NCHUNK, CK), jnp.int32),
            pltpu.VMEM((CK, H), jnp.float32),
            pltpu.VMEM((CK, 16), jnp.float32),
            pltpu.VMEM((125, H), jnp.float32),
            pltpu.VMEM_SHARED((N, H), jnp.float32),
            pltpu.VMEM_SHARED((N, 16), jnp.float32),
        ],
    )
    return kern(src3, dst3, h, as_, ad_)


def kernel(x, edge_index, W1, a1_src, a1_dst, b1, W2, a2_src, a2_dst, b2,
           Wl1, bl1, Wl2, bl2):
    loop = jnp.arange(N, dtype=edge_index.dtype)
    pad = jnp.zeros((EPAD - ETOT,), edge_index.dtype)
    src3 = jnp.concatenate([edge_index[0], loop, pad]).reshape(
        NTILES, NCHUNK, CK)
    dst3 = jnp.concatenate([edge_index[1], loop, pad]).reshape(
        NTILES, NCHUNK, CK)

    h1, as1, ad1 = _tc_a(x, W1, a1_src, a1_dst)
    acc1, den1 = _sc_edge_pass(src3, dst3, h1,
                               as1.reshape(N), ad1.reshape(N))
    h2, as2, ad2 = _tc_b(acc1, den1, b1, W2, a2_src, a2_dst)
    acc2, den2 = _sc_edge_pass(src3, dst3, h2,
                               as2.reshape(N), ad2.reshape(N))
    return _tc_c(acc2, den2, b2, Wl1, bl1, Wl2, bl2)


# speedup vs baseline: 39.4125x; 39.4125x over previous
import jax
import jax.numpy as jnp
from jax import lax
from jax.experimental import pallas as pl
from jax.experimental.pallas import tpu as pltpu
from jax.experimental.pallas import tpu_sc as plsc

N = 10000
E = 320000
H = 64
NEG_SLOPE = 0.2
ETOT = E + N

NTILES = 32
NCHUNK = 81
CK = 128
EPT = NCHUNK * CK
EPAD = NTILES * EPT
RPT = 1000
ZR = 200


_BN = 2000


def _tc_a_body(x_ref, w_ref, asv_ref, adv_ref, h_ref, as_ref, ad_ref):
    h = jnp.dot(x_ref[...], w_ref[...], preferred_element_type=jnp.float32)
    h_ref[...] = h
    as_ref[...] = jnp.dot(h, asv_ref[...], preferred_element_type=jnp.float32)
    ad_ref[...] = jnp.dot(h, adv_ref[...], preferred_element_type=jnp.float32)


def _tc_a(x, W, a_src, a_dst):
    din = x.shape[1]
    return pl.pallas_call(
        _tc_a_body,
        grid=(N // _BN,),
        in_specs=[
            pl.BlockSpec((_BN, din), lambda i: (i, 0)),
            pl.BlockSpec((din, H), lambda i: (0, 0)),
            pl.BlockSpec((H, 1), lambda i: (0, 0)),
            pl.BlockSpec((H, 1), lambda i: (0, 0)),
        ],
        out_specs=[
            pl.BlockSpec((_BN, H), lambda i: (i, 0)),
            pl.BlockSpec((_BN, 1), lambda i: (i, 0)),
            pl.BlockSpec((_BN, 1), lambda i: (i, 0)),
        ],
        out_shape=[
            jax.ShapeDtypeStruct((N, H), jnp.float32),
            jax.ShapeDtypeStruct((N, 1), jnp.float32),
            jax.ShapeDtypeStruct((N, 1), jnp.float32),
        ],
    )(x, W, a_src.reshape(H, 1), a_dst.reshape(H, 1))


def _tc_b_body(acc_ref, den_ref, b_ref, w_ref, asv_ref, adv_ref,
               h_ref, as_ref, ad_ref):
    a = acc_ref[0] + acc_ref[1]
    d = den_ref[0, :, 0:1] + den_ref[1, :, 0:1]
    hag = a / (d + 1e-16) + b_ref[...]
    hag = jnp.maximum(hag, 0.0)
    h = jnp.dot(hag, w_ref[...], preferred_element_type=jnp.float32)
    h_ref[...] = h
    as_ref[...] = jnp.dot(h, asv_ref[...], preferred_element_type=jnp.float32)
    ad_ref[...] = jnp.dot(h, adv_ref[...], preferred_element_type=jnp.float32)


def _tc_b(acc, den, b, W, a_src, a_dst):
    return pl.pallas_call(
        _tc_b_body,
        grid=(N // _BN,),
        in_specs=[
            pl.BlockSpec((2, _BN, H), lambda i: (0, i, 0)),
            pl.BlockSpec((2, _BN, 16), lambda i: (0, i, 0)),
            pl.BlockSpec((1, H), lambda i: (0, 0)),
            pl.BlockSpec((H, H), lambda i: (0, 0)),
            pl.BlockSpec((H, 1), lambda i: (0, 0)),
            pl.BlockSpec((H, 1), lambda i: (0, 0)),
        ],
        out_specs=[
            pl.BlockSpec((_BN, H), lambda i: (i, 0)),
            pl.BlockSpec((_BN, 1), lambda i: (i, 0)),
            pl.BlockSpec((_BN, 1), lambda i: (i, 0)),
        ],
        out_shape=[
            jax.ShapeDtypeStruct((N, H), jnp.float32),
            jax.ShapeDtypeStruct((N, 1), jnp.float32),
            jax.ShapeDtypeStruct((N, 1), jnp.float32),
        ],
    )(acc, den, b.reshape(1, H), W, a_src.reshape(H, 1), a_dst.reshape(H, 1))


def _tc_c_body(acc_ref, den_ref, b_ref, w1_ref, b1_ref, w2_ref, b2_ref,
               out_ref):
    a = acc_ref[0] + acc_ref[1]
    d = den_ref[0, :, 0:1] + den_ref[1, :, 0:1]
    hag = a / (d + 1e-16) + b_ref[...]
    hag = jnp.maximum(hag, 0.0)
    h = jnp.dot(hag, w1_ref[...], preferred_element_type=jnp.float32)
    h = jnp.maximum(h + b1_ref[...], 0.0)
    out_ref[...] = jnp.dot(h, w2_ref[...],
                           preferred_element_type=jnp.float32) + b2_ref[...]


def _tc_c(acc, den, b, Wl1, bl1, Wl2, bl2):
    return pl.pallas_call(
        _tc_c_body,
        grid=(N // _BN,),
        in_specs=[
            pl.BlockSpec((2, _BN, H), lambda i: (0, i, 0)),
            pl.BlockSpec((2, _BN, 16), lambda i: (0, i, 0)),
            pl.BlockSpec((1, H), lambda i: (0, 0)),
            pl.BlockSpec((H, H), lambda i: (0, 0)),
            pl.BlockSpec((1, H), lambda i: (0, 0)),
            pl.BlockSpec((H, H), lambda i: (0, 0)),
            pl.BlockSpec((1, H), lambda i: (0, 0)),
        ],
        out_specs=pl.BlockSpec((_BN, H), lambda i: (i, 0)),
        out_shape=jax.ShapeDtypeStruct((N, H), jnp.float32),
    )(acc, den, b.reshape(1, H), Wl1, bl1.reshape(1, H), Wl2,
      bl2.reshape(1, H))


def _sc_body(src_hbm, dst_hbm, h_hbm, as_hbm, ad_hbm,
             acc_out, den_out,
             as_v, ad_v, sidx_v, didx_v, rows_v, wrow_v, zbuf_v, zden_v,
             mbuf_v, acc_sp, den_sp):
    cid = lax.axis_index("c")
    sid = lax.axis_index("s")
    wid = cid * 16 + sid

    pltpu.sync_copy(as_hbm, as_v)
    pltpu.sync_copy(ad_hbm, ad_v)
    pltpu.sync_copy(src_hbm.at[wid], sidx_v)
    pltpu.sync_copy(dst_hbm.at[wid], didx_v)

    zero16 = jnp.zeros((16,), jnp.float32)

    @pl.loop(0, ZR)
    def _(i):
        for f in range(4):
            zbuf_v[i, pl.ds(16 * f, 16)] = zero16
        zden_v[i, :] = zero16

    @pl.when(sid < 10)
    def _():
        for r in range(RPT // ZR):
            pltpu.sync_copy(
                zbuf_v, acc_sp.at[pl.ds(sid * RPT + r * ZR, ZR)])
            pltpu.sync_copy(
                zden_v, den_sp.at[pl.ds(sid * RPT + r * ZR, ZR)])

    lane16 = lax.iota(jnp.int32, 16)

    def _vmax(ref):
        def body(i, m):
            return jnp.maximum(m, ref[pl.ds(16 * i, 16)])
        m = lax.fori_loop(0, N // 16, body,
                          jnp.full((16,), -jnp.inf, jnp.float32))
        for sh in (8, 4, 2, 1):
            mbuf_v[...] = m
            m = jnp.maximum(m, plsc.load_gather(
                mbuf_v, [jnp.bitwise_xor(lane16, sh)]))
        return m

    cmax = _vmax(as_v) + _vmax(ad_v)
    cshift = jnp.where(cmax >= 0.0, cmax, NEG_SLOPE * cmax)

    @pl.loop(0, CK)
    def _(i):
        wrow_v[i, :] = zero16

    plsc.subcore_barrier()

    lane = lax.iota(jnp.int32, 16)
    zlane = jnp.zeros((16,), jnp.int32)
    base = wid * EPT

    @pl.loop(0, NCHUNK)
    def _chunk(j):
        pltpu.sync_copy(h_hbm.at[sidx_v.at[j]], rows_v)

        for g in range(8):
            si = sidx_v[j, pl.ds(16 * g, 16)]
            di = didx_v[j, pl.ds(16 * g, 16)]
            e = plsc.load_gather(as_v, [si]) + plsc.load_gather(ad_v, [di])
            e = jnp.where(e >= 0.0, e, NEG_SLOPE * e)
            w = jnp.exp(e - cshift)
            eid = base + j * CK + 16 * g + lane
            w = jnp.where(eid < ETOT, w, 0.0)
            plsc.store_scatter(wrow_v, [16 * g + lane, zlane], w)
            for i in range(16):
                s = w[i]
                row = 16 * g + i
                for f in range(4):
                    rows_v[row, pl.ds(16 * f, 16)] = (
                        rows_v[row, pl.ds(16 * f, 16)] * s)

        pltpu.sync_copy(rows_v, acc_sp.at[didx_v.at[j]], add=True)
        pltpu.sync_copy(wrow_v, den_sp.at[didx_v.at[j]], add=True)

    plsc.subcore_barrier()

    @pl.when(sid < 10)
    def _():
        pltpu.sync_copy(acc_sp.at[pl.ds(sid * RPT, RPT)],
                        acc_out.at[cid, pl.ds(sid * RPT, RPT)])
        pltpu.sync_copy(den_sp.at[pl.ds(sid * RPT, RPT)],
                        den_out.at[cid, pl.ds(sid * RPT, RPT)])


def _sc_edge_pass(src3, dst3, h, as_, ad_):
    kern = pl.kernel(
        _sc_body,
        out_type=[
            jax.ShapeDtypeStruct((2, N, H), jnp.float32),
            jax.ShapeDtypeStruct((2, N, 16), jnp.float32),
        ],
        mesh=plsc.VectorSubcoreMesh(core_axis_name="c", subcore_axis_name="s"),
        compiler_params=pltpu.CompilerParams(
            needs_layout_passes=False, use_tc_tiling_on_sc=False),
        scratch_types=[
            pltpu.VMEM((N,), jnp.float32),
            pltpu.VMEM((N,), jnp.float32),
            pltpu.VMEM((NCHUNK, CK), jnp.int32),
            pltpu.VMEM((NCHUNK, CK), jnp.int32),
            pltpu.VMEM((CK, H), jnp.float32),
            pltpu.VMEM((CK, 16), jnp.float32),
            pltpu.VMEM((ZR, H), jnp.float32),
            pltpu.VMEM((ZR, 16), jnp.float32),
            pltpu.VMEM((16,), jnp.float32),
            pltpu.VMEM_SHARED((N, H), jnp.float32),
            pltpu.VMEM_SHARED((N, 16), jnp.float32),
        ],
    )
    return kern(src3, dst3, h, as_, ad_)


def kernel(x, edge_index, W1, a1_src, a1_dst, b1, W2, a2_src, a2_dst, b2,
           Wl1, bl1, Wl2, bl2):
    loop = jnp.arange(N, dtype=edge_index.dtype)
    pad = jnp.zeros((EPAD - ETOT,), edge_index.dtype)
    src3 = jnp.concatenate([edge_index[0], loop, pad]).reshape(
        NTILES, NCHUNK, CK)
    dst3 = jnp.concatenate([edge_index[1], loop, pad]).reshape(
        NTILES, NCHUNK, CK)

    h1, as1, ad1 = _tc_a(x, W1, a1_src, a1_dst)
    acc1, den1 = _sc_edge_pass(src3, dst3, h1,
                               as1.reshape(N), ad1.reshape(N))
    h2, as2, ad2 = _tc_b(acc1, den1, b1, W2, a2_src, a2_dst)
    acc2, den2 = _sc_edge_pass(src3, dst3, h2,
                               as2.reshape(N), ad2.reshape(N))
    return _tc_c(acc2, den2, b2, Wl1, bl1, Wl2, bl2)
